# baseline (speedup 1.0000x reference)
import functools

import jax
import jax.numpy as jnp
from jax import lax
from jax.experimental import pallas as pl
from jax.experimental.pallas import tpu as pltpu
from jax.experimental.pallas import tpu_sc as plsc

B = 16384
K = 32
NC = 2
NS = 16
L = 16
NW = NC * NS
BPW = B // NW
CHK = 16
NCHK = BPW // CHK
SLAB = 8
NSLAB = 125000
DEPTH = 3


def _body(uid_hbm, iid_hbm, pi3_hbm, eta3_hbm, out_hbm,
          uid_v, iid_v, out_v, stash_v,
          pi_0, pi_1, pi_2, eta_0, eta_1, eta_2, sem_0, sem_1, sem_2):
    wid = lax.axis_index("s") * NC + lax.axis_index("c")

    pltpu.sync_copy(uid_hbm.at[wid], uid_v)
    pltpu.sync_copy(iid_hbm.at[wid], iid_v)

    last_lane = lax.iota(jnp.int32, L) * L + (L - 1)
    bufs = [(pi_0, eta_0, sem_0), (pi_1, eta_1, sem_1), (pi_2, eta_2, sem_2)]

    def fire(c, pi_buf, eta_buf, sem):
        uvec = uid_v[pl.ds(c * CHK, CHK)]
        tvec = iid_v[pl.ds(c * CHK, CHK)]
        for j in range(CHK):
            pltpu.async_copy(
                pi3_hbm.at[uvec[j] >> 3], pi_buf.at[j], sem)
            pltpu.async_copy(
                eta3_hbm.at[tvec[j] >> 3], eta_buf.at[j], sem)

    def drain(pi_buf, eta_buf, sem):
        pltpu.make_async_copy(pi3_hbm.at[pl.ds(0, CHK)], pi_buf, sem).wait()
        pltpu.make_async_copy(eta3_hbm.at[pl.ds(0, CHK)], eta_buf, sem).wait()

    def compute(c, pi_buf, eta_buf):
        uvec = uid_v[pl.ds(c * CHK, CHK)]
        tvec = iid_v[pl.ds(c * CHK, CHK)]
        for j in range(CHK):
            r = uvec[j] & (SLAB - 1)
            s = tvec[j] & (SLAB - 1)
            v = (pi_buf[j, r, pl.ds(0, L)] * eta_buf[j, s, pl.ds(0, L)]
                 + pi_buf[j, r, pl.ds(L, L)] * eta_buf[j, s, pl.ds(L, L)])
            stash_v[pl.ds(j * L, L)] = plsc.cumsum(v)
        sums = plsc.load_gather(stash_v, [last_lane])
        out_v[pl.ds(c * CHK, CHK)] = 1.0 - jnp.exp(-sums)

    fire(0, *bufs[0])
    fire(1, *bufs[1])

    def triple(p, carry):
        for q in range(DEPTH):
            c = p * DEPTH + q
            fire(c + 2, *bufs[(q + 2) % DEPTH])
            pb, eb, sem = bufs[q]
            drain(pb, eb, sem)
            compute(c, pb, eb)
        return carry

    lax.fori_loop(0, (NCHK - 2) // DEPTH, triple, 0)

    for c in range(NCHK - 2, NCHK):
        pb, eb, sem = bufs[c % DEPTH]
        drain(pb, eb, sem)
        compute(c, pb, eb)

    pltpu.sync_copy(out_v, out_hbm.at[pl.ds(wid * BPW, BPW)])


_pf = functools.partial(
    pl.kernel,
    mesh=plsc.VectorSubcoreMesh(core_axis_name="c", subcore_axis_name="s"),
    out_type=jax.ShapeDtypeStruct((B,), jnp.float32),
    compiler_params=pltpu.CompilerParams(needs_layout_passes=False, allow_input_fusion=[0, 1, 2, 3]),
    scratch_types=[
        pltpu.VMEM((BPW,), jnp.int32),
        pltpu.VMEM((BPW,), jnp.int32),
        pltpu.VMEM((BPW,), jnp.float32),
        pltpu.VMEM((CHK * L,), jnp.float32),
        pltpu.VMEM((CHK, SLAB, K), jnp.float32),
        pltpu.VMEM((CHK, SLAB, K), jnp.float32),
        pltpu.VMEM((CHK, SLAB, K), jnp.float32),
        pltpu.VMEM((CHK, SLAB, K), jnp.float32),
        pltpu.VMEM((CHK, SLAB, K), jnp.float32),
        pltpu.VMEM((CHK, SLAB, K), jnp.float32),
        pltpu.SemaphoreType.DMA,
        pltpu.SemaphoreType.DMA,
        pltpu.SemaphoreType.DMA,
    ],
)(_body)


def kernel(user_ids, item_ids, pi, eta):
    uid = user_ids.astype(jnp.int32).reshape(NW, BPW)
    iid = item_ids.astype(jnp.int32).reshape(NW, BPW)
    pi3 = pi.reshape(NSLAB, SLAB, K)
    eta3 = eta.reshape(NSLAB, SLAB, K)
    return _pf(uid, iid, pi3, eta3)


# speedup vs baseline: 1.0023x; 1.0023x over previous
import functools

import jax
import jax.numpy as jnp
from jax import lax
from jax.experimental import pallas as pl
from jax.experimental.pallas import tpu as pltpu
from jax.experimental.pallas import tpu_sc as plsc

B = 16384
K = 32
NC = 2
NS = 16
L = 16
NW = NC * NS
BPW = B // NW
CHK = 16
NCHK = BPW // CHK
SLAB = 8
NSLAB = 125000
DEPTH = 3


def _body(uid_hbm, iid_hbm, pi3_hbm, eta3_hbm, out_hbm,
          uid_v, iid_v, out_v, stash_v,
          pi_0, pi_1, pi_2, eta_0, eta_1, eta_2, sem_0, sem_1, sem_2):
    wid = lax.axis_index("s") * NC + lax.axis_index("c")

    pltpu.sync_copy(uid_hbm.at[wid], uid_v)
    pltpu.sync_copy(iid_hbm.at[wid], iid_v)

    last_lane = lax.iota(jnp.int32, L) * L + (L - 1)
    bufs = [(pi_0, eta_0, sem_0), (pi_1, eta_1, sem_1), (pi_2, eta_2, sem_2)]

    def fire(c, pi_buf, eta_buf, sem):
        uvec = uid_v[pl.ds(c * CHK, CHK)]
        tvec = iid_v[pl.ds(c * CHK, CHK)]
        for j in range(CHK):
            pltpu.async_copy(
                pi3_hbm.at[uvec[j] >> 3], pi_buf.at[j], sem)
            pltpu.async_copy(
                eta3_hbm.at[tvec[j] >> 3], eta_buf.at[j], sem)

    def drain(pi_buf, eta_buf, sem):
        pltpu.make_async_copy(pi3_hbm.at[pl.ds(0, CHK)], pi_buf, sem).wait()
        pltpu.make_async_copy(eta3_hbm.at[pl.ds(0, CHK)], eta_buf, sem).wait()

    def compute(c, pi_buf, eta_buf):
        uvec = uid_v[pl.ds(c * CHK, CHK)]
        tvec = iid_v[pl.ds(c * CHK, CHK)]
        for j in range(CHK):
            r = uvec[j] & (SLAB - 1)
            s = tvec[j] & (SLAB - 1)
            v = (pi_buf[j, r, pl.ds(0, L)] * eta_buf[j, s, pl.ds(0, L)]
                 + pi_buf[j, r, pl.ds(L, L)] * eta_buf[j, s, pl.ds(L, L)])
            stash_v[pl.ds(j * L, L)] = plsc.cumsum(v)
        sums = plsc.load_gather(stash_v, [last_lane])
        out_v[pl.ds(c * CHK, CHK)] = 1.0 - jnp.exp(-sums)

    fire(0, *bufs[0])
    fire(1, *bufs[1])

    def triple(p, carry):
        for q in range(DEPTH):
            c = p * DEPTH + q
            fire(c + 2, *bufs[(q + 2) % DEPTH])
            pb, eb, sem = bufs[q]
            drain(pb, eb, sem)
            compute(c, pb, eb)
        return carry

    lax.fori_loop(0, (NCHK - 2) // DEPTH, triple, 0)

    for c in range(NCHK - 2, NCHK):
        pb, eb, sem = bufs[c % DEPTH]
        drain(pb, eb, sem)
        compute(c, pb, eb)

    pltpu.sync_copy(out_v, out_hbm.at[pl.ds(wid * BPW, BPW)])


_pf = functools.partial(
    pl.kernel,
    mesh=plsc.VectorSubcoreMesh(core_axis_name="c", subcore_axis_name="s"),
    out_type=jax.ShapeDtypeStruct((B,), jnp.float32),
    compiler_params=pltpu.CompilerParams(needs_layout_passes=False),
    scratch_types=[
        pltpu.VMEM((BPW,), jnp.int32),
        pltpu.VMEM((BPW,), jnp.int32),
        pltpu.VMEM((BPW,), jnp.float32),
        pltpu.VMEM((CHK * L,), jnp.float32),
        pltpu.VMEM((CHK, SLAB, K), jnp.float32),
        pltpu.VMEM((CHK, SLAB, K), jnp.float32),
        pltpu.VMEM((CHK, SLAB, K), jnp.float32),
        pltpu.VMEM((CHK, SLAB, K), jnp.float32),
        pltpu.VMEM((CHK, SLAB, K), jnp.float32),
        pltpu.VMEM((CHK, SLAB, K), jnp.float32),
        pltpu.SemaphoreType.DMA,
        pltpu.SemaphoreType.DMA,
        pltpu.SemaphoreType.DMA,
    ],
)(_body)


def kernel(user_ids, item_ids, pi, eta):
    uid = user_ids.astype(jnp.int32).reshape(NW, BPW)
    iid = item_ids.astype(jnp.int32).reshape(NW, BPW)
    pi3 = pi.reshape(NSLAB, SLAB, K)
    eta3 = eta.reshape(NSLAB, SLAB, K)
    return _pf(uid, iid, pi3, eta3)
